# baseline (speedup 1.0000x reference)
import functools

import jax
import jax.numpy as jnp
from jax import lax
from jax.experimental import pallas as pl
from jax.experimental.pallas import tpu as pltpu
from jax.experimental.pallas import tpu_sc as plsc

_CLASS_MAP = (19, 19, 19, 19, 19, 19, 19, 0, 1, 19, 19, 2, 3, 4, 19, 19, 19, 5,
              19, 6, 7, 8, 9, 10, 11, 12, 13, 14, 15, 19, 19, 16, 17, 18, 19)
_TABLE_PAD = 48

_B, _H, _W = 16, 512, 512
_NC, _NS, _L = 2, 16, 16
_NW = _NC * _NS
_ROWS_W = _B * _H // _NW
_BLK_R = 32
_NBLK = _ROWS_W // _BLK_R
_BLK = _BLK_R * _W

_PADDED_MAP = _CLASS_MAP + (19,) * (_TABLE_PAD - len(_CLASS_MAP))

_mesh = plsc.VectorSubcoreMesh(core_axis_name="c", subcore_axis_name="s",
                               num_cores=_NC, num_subcores=_NS)


@functools.partial(
    pl.kernel,
    out_type=jax.ShapeDtypeStruct((_B, _H, _W), jnp.int32),
    mesh=_mesh,
    compiler_params=pltpu.CompilerParams(needs_layout_passes=False,
                                         skip_device_barrier=True),
    scratch_types=[
        pltpu.VMEM((_TABLE_PAD,), jnp.int32),
        pltpu.VMEM((2, _BLK_R, _W), jnp.int32),
        pltpu.VMEM((2, _BLK_R, _W), jnp.int32),
        pltpu.SemaphoreType.DMA,
        pltpu.SemaphoreType.DMA,
        pltpu.SemaphoreType.DMA,
        pltpu.SemaphoreType.DMA,
    ],
)
def _remap(tree_hbm, out_hbm, table_v, ibuf, obuf, si0, si1, so0, so1):
  wid = lax.axis_index("s") * _NC + lax.axis_index("c")
  img = wid // 2
  row0 = (wid % 2) * _ROWS_W
  si = (si0, si1)
  so = (so0, so1)

  for v in range(_TABLE_PAD // _L):
    x = lax.iota(jnp.int32, _L) + v * _L
    off = (7 + 2 * (x >= 11).astype(jnp.int32)
           + 3 * (x >= 17).astype(jnp.int32)
           + (x >= 19).astype(jnp.int32)
           + 2 * (x >= 31).astype(jnp.int32))
    valid = ((x >= 7) & (x <= 33)
             & ~((x >= 9) & (x <= 10))
             & ~((x >= 14) & (x <= 16))
             & (x != 18)
             & ~((x >= 29) & (x <= 30)))
    table_v[pl.ds(v * _L, _L)] = jnp.where(valid, x - off, 19)

  cp_in = [None, None]
  cp_out = [None, None]
  cp_in[0] = pltpu.async_copy(
      tree_hbm.at[img, pl.ds(row0, _BLK_R)], ibuf.at[0], si[0])
  for b in range(_NBLK):
    s = b % 2
    if b + 1 < _NBLK:
      cp_in[(b + 1) % 2] = pltpu.async_copy(
          tree_hbm.at[img, pl.ds(row0 + (b + 1) * _BLK_R, _BLK_R)],
          ibuf.at[(b + 1) % 2], si[(b + 1) % 2])
    cp_in[s].wait()
    if cp_out[s] is not None:
      cp_out[s].wait()

    @plsc.parallel_loop(0, _BLK, _L, unroll=16)
    def vec_body(i):
      r = i >> 9
      c = i & (_W - 1)
      idx = ibuf[s, r, pl.ds(c, _L)]
      obuf[s, r, pl.ds(c, _L)] = plsc.load_gather(table_v, [idx])

    cp_out[s] = pltpu.async_copy(
        obuf.at[s], out_hbm.at[img, pl.ds(row0 + b * _BLK_R, _BLK_R)], so[s])
  cp_out[(_NBLK - 2) % 2].wait()
  cp_out[(_NBLK - 1) % 2].wait()


def kernel(tree):
  return _remap(tree.astype(jnp.int32)).astype(tree.dtype)


# speedup vs baseline: 1.0205x; 1.0205x over previous
import functools

import jax
import jax.numpy as jnp
from jax import lax
from jax.experimental import pallas as pl
from jax.experimental.pallas import tpu as pltpu
from jax.experimental.pallas import tpu_sc as plsc

_CLASS_MAP = (19, 19, 19, 19, 19, 19, 19, 0, 1, 19, 19, 2, 3, 4, 19, 19, 19, 5,
              19, 6, 7, 8, 9, 10, 11, 12, 13, 14, 15, 19, 19, 16, 17, 18, 19)
_TABLE_PAD = 48

_B, _H, _W = 16, 512, 512
_NC, _NS, _L = 2, 16, 16
_NW = _NC * _NS
_ROWS_W = _B * _H // _NW
_BLK_R = 32
_NBLK = _ROWS_W // _BLK_R
_BLK = _BLK_R * _W

_PADDED_MAP = _CLASS_MAP + (19,) * (_TABLE_PAD - len(_CLASS_MAP))

_mesh = plsc.VectorSubcoreMesh(core_axis_name="c", subcore_axis_name="s",
                               num_cores=_NC, num_subcores=_NS)


@functools.partial(
    pl.kernel,
    out_type=jax.ShapeDtypeStruct((_B, _H, _W), jnp.int32),
    mesh=_mesh,
    compiler_params=pltpu.CompilerParams(needs_layout_passes=False),
    scratch_types=[
        pltpu.VMEM((_TABLE_PAD,), jnp.int32),
        pltpu.VMEM((2, _BLK_R, _W), jnp.int32),
        pltpu.VMEM((2, _BLK_R, _W), jnp.int32),
        pltpu.SemaphoreType.DMA,
        pltpu.SemaphoreType.DMA,
        pltpu.SemaphoreType.DMA,
        pltpu.SemaphoreType.DMA,
    ],
)
def _remap(tree_hbm, out_hbm, table_v, ibuf, obuf, si0, si1, so0, so1):
  wid = lax.axis_index("s") * _NC + lax.axis_index("c")
  img = wid // 2
  row0 = (wid % 2) * _ROWS_W
  si = (si0, si1)
  so = (so0, so1)

  for v in range(_TABLE_PAD // _L):
    x = lax.iota(jnp.int32, _L) + v * _L
    off = (7 + 2 * (x >= 11).astype(jnp.int32)
           + 3 * (x >= 17).astype(jnp.int32)
           + (x >= 19).astype(jnp.int32)
           + 2 * (x >= 31).astype(jnp.int32))
    valid = ((x >= 7) & (x <= 33)
             & ~((x >= 9) & (x <= 10))
             & ~((x >= 14) & (x <= 16))
             & (x != 18)
             & ~((x >= 29) & (x <= 30)))
    table_v[pl.ds(v * _L, _L)] = jnp.where(valid, x - off, 19)

  cp_in = [None, None]
  cp_out = [None, None]
  cp_in[0] = pltpu.async_copy(
      tree_hbm.at[img, pl.ds(row0, _BLK_R)], ibuf.at[0], si[0])
  for b in range(_NBLK):
    s = b % 2
    if b + 1 < _NBLK:
      cp_in[(b + 1) % 2] = pltpu.async_copy(
          tree_hbm.at[img, pl.ds(row0 + (b + 1) * _BLK_R, _BLK_R)],
          ibuf.at[(b + 1) % 2], si[(b + 1) % 2])
    cp_in[s].wait()
    if cp_out[s] is not None:
      cp_out[s].wait()

    @plsc.parallel_loop(0, _BLK, _L, unroll=8)
    def vec_body(i):
      r = i >> 9
      c = i & (_W - 1)
      idx = ibuf[s, r, pl.ds(c, _L)]
      obuf[s, r, pl.ds(c, _L)] = plsc.load_gather(table_v, [idx])

    cp_out[s] = pltpu.async_copy(
        obuf.at[s], out_hbm.at[img, pl.ds(row0 + b * _BLK_R, _BLK_R)], so[s])
  cp_out[(_NBLK - 2) % 2].wait()
  cp_out[(_NBLK - 1) % 2].wait()


def kernel(tree):
  return _remap(tree.astype(jnp.int32)).astype(tree.dtype)
